# baseline (speedup 1.0000x reference)
import functools
import jax
import jax.numpy as jnp
from jax import lax
from jax.experimental import pallas as pl
from jax.experimental.pallas import tpu as pltpu

N = 10000
E = 160000
H = 128
EBLK = 2000
NEB = E // EBLK


def _onehot(idx_col, k):
    b = idx_col.shape[0]
    cols = lax.broadcasted_iota(jnp.int32, (b, k), 1)
    return (idx_col == cols).astype(jnp.float32)


def _tupleinit_body(xi_ref, xj_ref, tf_ref, xemb_ref, w0_ref, b0_ref,
                    w1_ref, b1_ref, tfemb_ref, out_ref):
    t0 = jnp.dot(xemb_ref[...], w0_ref[...],
                 preferred_element_type=jnp.float32) + b0_ref[...]
    t1 = jnp.dot(xemb_ref[...], w1_ref[...],
                 preferred_element_type=jnp.float32) + b1_ref[...]
    a = jnp.dot(_onehot(xi_ref[...], 32), t0,
                preferred_element_type=jnp.float32)
    b = jnp.dot(_onehot(xj_ref[...], 32), t1,
                preferred_element_type=jnp.float32)
    c = jnp.dot(_onehot(tf_ref[...], 16), tfemb_ref[...],
                preferred_element_type=jnp.float32)
    out_ref[...] = a * b * c


def _layer_body(g_ref, ea_ref, xv_ref, eaemb_ref, w1_ref, b1_ref,
                w2_ref, b2_ref, out_ref):
    eaw = jnp.dot(_onehot(ea_ref[...], 16), eaemb_ref[...],
                  preferred_element_type=jnp.float32)
    m = g_ref[...] * eaw
    h = jnp.maximum(jnp.dot(m, w1_ref[...],
                            preferred_element_type=jnp.float32)
                    + b1_ref[...], 0.0)
    tx = jnp.dot(h, w2_ref[...],
                 preferred_element_type=jnp.float32) + b2_ref[...]
    out_ref[...] = xv_ref[...] + tx


def _pred_body(hg_ref, pw_ref, pb_ref, out_ref):
    out_ref[...] = jnp.dot(hg_ref[...], pw_ref[...],
                           preferred_element_type=jnp.float32) + pb_ref[...]


def _eblk_spec():
    return pl.BlockSpec((EBLK, 1), lambda i: (i, 0))


def _erow_spec():
    return pl.BlockSpec((EBLK, H), lambda i: (i, 0))


def _full(shape):
    return pl.BlockSpec(shape, lambda i: tuple(0 for _ in shape))


def kernel(x, edge_index, edge_attr, tuple_index, tuple_feat, batch,
           num_graphs, x_emb, ea_emb, tf_emb, W0, b0, W1, b1,
           convW1, convb1, convW2, convb2, predW, predb):
    G = 128
    ti = tuple_index[0]
    tj = tuple_index[1]
    xi = jnp.take(x, ti, axis=0).astype(jnp.int32).reshape(E, 1)
    xj = jnp.take(x, tj, axis=0).astype(jnp.int32).reshape(E, 1)
    tfc = tuple_feat.astype(jnp.int32).reshape(E, 1)
    eac = edge_attr.astype(jnp.int32).reshape(E, 1)
    b0r = b0.reshape(1, H)
    b1r = b1.reshape(1, H)

    Xval = pl.pallas_call(
        _tupleinit_body,
        grid=(NEB,),
        in_specs=[_eblk_spec(), _eblk_spec(), _eblk_spec(),
                  _full((32, H)), _full((H, H)), _full((1, H)),
                  _full((H, H)), _full((1, H)), _full((16, H))],
        out_specs=_erow_spec(),
        out_shape=jax.ShapeDtypeStruct((E, H), jnp.float32),
    )(xi, xj, tfc, x_emb, W0, b0r, W1, b1r, tf_emb)

    layer_call = pl.pallas_call(
        _layer_body,
        grid=(NEB,),
        in_specs=[_erow_spec(), _eblk_spec(), _erow_spec(),
                  _full((16, H)), _full((H, H)), _full((1, H)),
                  _full((H, H)), _full((1, H))],
        out_specs=_erow_spec(),
        out_shape=jax.ShapeDtypeStruct((E, H), jnp.float32),
    )

    for l in range(convW1.shape[0]):
        agg = jax.ops.segment_sum(Xval, tj, num_segments=N)
        g = jnp.take(agg, ti, axis=0)
        Xval = layer_call(g, eac, Xval, ea_emb, convW1[l],
                          convb1[l].reshape(1, H), convW2[l],
                          convb2[l].reshape(1, H))

    xn = jax.ops.segment_max(Xval, ti, num_segments=N)
    xn = jnp.where(jnp.isfinite(xn), xn, 0.0)

    seg_valid = (jnp.arange(G)[:, None] < num_graphs).astype(xn.dtype)
    hsum = jax.ops.segment_sum(xn, batch, num_segments=G) * seg_valid
    cnt = jax.ops.segment_sum(jnp.ones((N, 1), dtype=xn.dtype), batch,
                              num_segments=G) * seg_valid
    h_graph = hsum / jnp.maximum(cnt, 1.0)

    out = pl.pallas_call(
        _pred_body,
        in_specs=[pl.BlockSpec((G, H), lambda: (0, 0)),
                  pl.BlockSpec((H, 1), lambda: (0, 0)),
                  pl.BlockSpec((1, 1), lambda: (0, 0))],
        out_specs=pl.BlockSpec((G, 1), lambda: (0, 0)),
        out_shape=jax.ShapeDtypeStruct((G, 1), jnp.float32),
    )(h_graph, predW, predb.reshape(1, 1))
    return out


# speedup vs baseline: 1.2236x; 1.2236x over previous
import functools
import jax
import jax.numpy as jnp
from jax import lax
from jax.experimental import pallas as pl
from jax.experimental.pallas import tpu as pltpu
from jax.experimental.pallas import tpu_sc as plsc

N = 10000
E = 160000
H = 128
EBLK = 2000
NEB = E // EBLK

NW = 32
EPW = E // NW
GC = 200
NGR = EPW // GC

_sc_mesh = plsc.VectorSubcoreMesh(core_axis_name="c", subcore_axis_name="s")


@functools.partial(
    pl.kernel,
    out_type=jax.ShapeDtypeStruct((E, H), jnp.float32),
    mesh=_sc_mesh,
    scratch_types=[
        pltpu.VMEM((GC,), jnp.int32),
        pltpu.VMEM((GC, H), jnp.float32),
        pltpu.SemaphoreType.DMA,
    ],
)
def _sc_gather_rows(idx_hbm, table_hbm, out_hbm, idx_v, rows_v, sem):
    wid = lax.axis_index("s") * 2 + lax.axis_index("c")
    base = wid * EPW

    def body(r, carry):
        off = base + r * GC
        pltpu.sync_copy(idx_hbm.at[pl.ds(off, GC)], idx_v)
        pltpu.async_copy(table_hbm.at[idx_v], rows_v, sem).wait()
        pltpu.sync_copy(rows_v, out_hbm.at[pl.ds(off, GC)])
        return carry

    lax.fori_loop(0, NGR, body, 0)


def _onehot(idx_col, k):
    b = idx_col.shape[0]
    cols = lax.broadcasted_iota(jnp.int32, (b, k), 1)
    return (idx_col == cols).astype(jnp.float32)


def _tupleinit_body(xi_ref, xj_ref, tf_ref, xemb_ref, w0_ref, b0_ref,
                    w1_ref, b1_ref, tfemb_ref, out_ref):
    t0 = jnp.dot(xemb_ref[...], w0_ref[...],
                 preferred_element_type=jnp.float32) + b0_ref[...]
    t1 = jnp.dot(xemb_ref[...], w1_ref[...],
                 preferred_element_type=jnp.float32) + b1_ref[...]
    a = jnp.dot(_onehot(xi_ref[...], 32), t0,
                preferred_element_type=jnp.float32)
    b = jnp.dot(_onehot(xj_ref[...], 32), t1,
                preferred_element_type=jnp.float32)
    c = jnp.dot(_onehot(tf_ref[...], 16), tfemb_ref[...],
                preferred_element_type=jnp.float32)
    out_ref[...] = a * b * c


def _layer_body(g_ref, ea_ref, xv_ref, eaemb_ref, w1_ref, b1_ref,
                w2_ref, b2_ref, out_ref):
    eaw = jnp.dot(_onehot(ea_ref[...], 16), eaemb_ref[...],
                  preferred_element_type=jnp.float32)
    m = g_ref[...] * eaw
    h = jnp.maximum(jnp.dot(m, w1_ref[...],
                            preferred_element_type=jnp.float32)
                    + b1_ref[...], 0.0)
    tx = jnp.dot(h, w2_ref[...],
                 preferred_element_type=jnp.float32) + b2_ref[...]
    out_ref[...] = xv_ref[...] + tx


def _pred_body(hg_ref, pw_ref, pb_ref, out_ref):
    out_ref[...] = jnp.dot(hg_ref[...], pw_ref[...],
                           preferred_element_type=jnp.float32) + pb_ref[...]


def _eblk_spec():
    return pl.BlockSpec((EBLK, 1), lambda i: (i, 0))


def _erow_spec():
    return pl.BlockSpec((EBLK, H), lambda i: (i, 0))


def _full(shape):
    return pl.BlockSpec(shape, lambda i: tuple(0 for _ in shape))


def kernel(x, edge_index, edge_attr, tuple_index, tuple_feat, batch,
           num_graphs, x_emb, ea_emb, tf_emb, W0, b0, W1, b1,
           convW1, convb1, convW2, convb2, predW, predb):
    G = 128
    ti = tuple_index[0]
    tj = tuple_index[1]
    xi = jnp.take(x, ti, axis=0).astype(jnp.int32).reshape(E, 1)
    xj = jnp.take(x, tj, axis=0).astype(jnp.int32).reshape(E, 1)
    tfc = tuple_feat.astype(jnp.int32).reshape(E, 1)
    eac = edge_attr.astype(jnp.int32).reshape(E, 1)
    b0r = b0.reshape(1, H)
    b1r = b1.reshape(1, H)

    Xval = pl.pallas_call(
        _tupleinit_body,
        grid=(NEB,),
        in_specs=[_eblk_spec(), _eblk_spec(), _eblk_spec(),
                  _full((32, H)), _full((H, H)), _full((1, H)),
                  _full((H, H)), _full((1, H)), _full((16, H))],
        out_specs=_erow_spec(),
        out_shape=jax.ShapeDtypeStruct((E, H), jnp.float32),
    )(xi, xj, tfc, x_emb, W0, b0r, W1, b1r, tf_emb)

    layer_call = pl.pallas_call(
        _layer_body,
        grid=(NEB,),
        in_specs=[_erow_spec(), _eblk_spec(), _erow_spec(),
                  _full((16, H)), _full((H, H)), _full((1, H)),
                  _full((H, H)), _full((1, H))],
        out_specs=_erow_spec(),
        out_shape=jax.ShapeDtypeStruct((E, H), jnp.float32),
    )

    ti32 = ti.astype(jnp.int32)
    for l in range(convW1.shape[0]):
        agg = jax.ops.segment_sum(Xval, tj, num_segments=N)
        g = _sc_gather_rows(ti32, agg)
        Xval = layer_call(g, eac, Xval, ea_emb, convW1[l],
                          convb1[l].reshape(1, H), convW2[l],
                          convb2[l].reshape(1, H))

    xn = jax.ops.segment_max(Xval, ti, num_segments=N)
    xn = jnp.where(jnp.isfinite(xn), xn, 0.0)

    seg_valid = (jnp.arange(G)[:, None] < num_graphs).astype(xn.dtype)
    hsum = jax.ops.segment_sum(xn, batch, num_segments=G) * seg_valid
    cnt = jax.ops.segment_sum(jnp.ones((N, 1), dtype=xn.dtype), batch,
                              num_segments=G) * seg_valid
    h_graph = hsum / jnp.maximum(cnt, 1.0)

    out = pl.pallas_call(
        _pred_body,
        in_specs=[pl.BlockSpec((G, H), lambda: (0, 0)),
                  pl.BlockSpec((H, 1), lambda: (0, 0)),
                  pl.BlockSpec((1, 1), lambda: (0, 0))],
        out_specs=pl.BlockSpec((G, 1), lambda: (0, 0)),
        out_shape=jax.ShapeDtypeStruct((G, 1), jnp.float32),
    )(h_graph, predW, predb.reshape(1, 1))
    return out


# speedup vs baseline: 1.3023x; 1.0644x over previous
import functools
import jax
import jax.numpy as jnp
from jax import lax
from jax.experimental import pallas as pl
from jax.experimental.pallas import tpu as pltpu
from jax.experimental.pallas import tpu_sc as plsc

N = 10000
E = 160000
H = 128
EBLK = 2000
NEB = E // EBLK

NW = 32
EPW = E // NW
GC = 200
NGR = EPW // GC

_sc_mesh = plsc.VectorSubcoreMesh(core_axis_name="c", subcore_axis_name="s")


@functools.partial(
    pl.kernel,
    out_type=jax.ShapeDtypeStruct((E, H), jnp.float32),
    mesh=_sc_mesh,
    scratch_types=[
        pltpu.VMEM((GC,), jnp.int32),
        pltpu.VMEM((GC, H), jnp.float32),
        pltpu.SemaphoreType.DMA,
    ],
)
def _sc_gather_rows(idx_hbm, table_hbm, out_hbm, idx_v, rows_v, sem):
    wid = lax.axis_index("s") * 2 + lax.axis_index("c")
    base = wid * EPW

    def body(r, carry):
        off = base + r * GC
        pltpu.sync_copy(idx_hbm.at[pl.ds(off, GC)], idx_v)
        pltpu.async_copy(table_hbm.at[idx_v], rows_v, sem).wait()
        pltpu.sync_copy(rows_v, out_hbm.at[pl.ds(off, GC)])
        return carry

    lax.fori_loop(0, NGR, body, 0)


NPAD = 10240
TROWS = NPAD // 16
ZR = 160


@functools.partial(
    pl.kernel,
    out_type=jax.ShapeDtypeStruct((2, NPAD, H), jnp.float32),
    mesh=_sc_mesh,
    scratch_types=[
        pltpu.VMEM((GC,), jnp.int32),
        pltpu.VMEM((GC, H), jnp.float32),
        pltpu.VMEM((ZR, H), jnp.float32),
        pltpu.VMEM_SHARED((NPAD, H), jnp.float32),
        pltpu.SemaphoreType.DMA,
    ],
)
def _sc_segsum(idx_hbm, xval_hbm, out_hbm, idx_v, rows_v, zb_v, table_s, sem):
    c = lax.axis_index("c")
    s = lax.axis_index("s")
    wid = s * 2 + c
    base = wid * EPW

    def zrow(i, carry):
        def zcol(j, carry2):
            zb_v[i, pl.ds(j * 16, 16)] = jnp.zeros((16,), jnp.float32)
            return carry2
        return lax.fori_loop(0, H // 16, zcol, carry)

    lax.fori_loop(0, ZR, zrow, 0)

    def zdma(k, carry):
        pltpu.sync_copy(zb_v, table_s.at[pl.ds(s * TROWS + k * ZR, ZR)])
        return carry

    lax.fori_loop(0, TROWS // ZR, zdma, 0)
    plsc.subcore_barrier()

    def body(r, carry):
        off = base + r * GC
        pltpu.sync_copy(idx_hbm.at[pl.ds(off, GC)], idx_v)
        pltpu.sync_copy(xval_hbm.at[pl.ds(off, GC)], rows_v)
        pltpu.sync_copy(rows_v, table_s.at[idx_v], add=True)
        return carry

    lax.fori_loop(0, NGR, body, 0)
    plsc.subcore_barrier()

    pltpu.sync_copy(table_s.at[pl.ds(s * TROWS, TROWS)],
                    out_hbm.at[c, pl.ds(s * TROWS, TROWS)])


@functools.partial(
    pl.kernel,
    out_type=jax.ShapeDtypeStruct((E, H), jnp.float32),
    mesh=_sc_mesh,
    scratch_types=[
        pltpu.VMEM((GC,), jnp.int32),
        pltpu.VMEM((GC,), jnp.int32),
        pltpu.VMEM((GC,), jnp.int32),
        pltpu.VMEM((GC, H), jnp.float32),
        pltpu.VMEM((GC, H), jnp.float32),
        pltpu.VMEM((GC, H), jnp.float32),
        pltpu.SemaphoreType.DMA,
    ],
)
def _sc_tupleinit(xi_hbm, xj_hbm, tf_hbm, t0_hbm, t1_hbm, tfe_hbm, out_hbm,
                  i0_v, i1_v, i2_v, r0_v, r1_v, r2_v, sem):
    wid = lax.axis_index("s") * 2 + lax.axis_index("c")
    base = wid * EPW

    def body(r, carry):
        off = base + r * GC
        pltpu.sync_copy(xi_hbm.at[pl.ds(off, GC)], i0_v)
        pltpu.sync_copy(xj_hbm.at[pl.ds(off, GC)], i1_v)
        pltpu.sync_copy(tf_hbm.at[pl.ds(off, GC)], i2_v)
        pltpu.async_copy(t0_hbm.at[i0_v], r0_v, sem).wait()
        pltpu.async_copy(t1_hbm.at[i1_v], r1_v, sem).wait()
        pltpu.async_copy(tfe_hbm.at[i2_v], r2_v, sem).wait()

        def mrow(i, c1):
            def mcol(j, c2):
                s = pl.ds(j * 16, 16)
                r0_v[i, s] = r0_v[i, s] * r1_v[i, s] * r2_v[i, s]
                return c2
            return lax.fori_loop(0, H // 16, mcol, c1)

        lax.fori_loop(0, GC, mrow, 0)
        pltpu.sync_copy(r0_v, out_hbm.at[pl.ds(off, GC)])
        return carry

    lax.fori_loop(0, NGR, body, 0)


def _layer_body(g0_ref, g1_ref, ea_ref, xv_ref, w1_ref, b1_ref,
                w2_ref, b2_ref, out_ref):
    m = (g0_ref[...] + g1_ref[...]) * ea_ref[...]
    h = jnp.maximum(jnp.dot(m, w1_ref[...],
                            preferred_element_type=jnp.float32)
                    + b1_ref[...], 0.0)
    tx = jnp.dot(h, w2_ref[...],
                 preferred_element_type=jnp.float32) + b2_ref[...]
    out_ref[...] = xv_ref[...] + tx


def _pred_body(hg_ref, pw_ref, pb_ref, out_ref):
    out_ref[...] = jnp.dot(hg_ref[...], pw_ref[...],
                           preferred_element_type=jnp.float32) + pb_ref[...]


def _eblk_spec():
    return pl.BlockSpec((EBLK, 1), lambda i: (i, 0))


def _erow_spec():
    return pl.BlockSpec((EBLK, H), lambda i: (i, 0))


def _full(shape):
    return pl.BlockSpec(shape, lambda i: tuple(0 for _ in shape))


def kernel(x, edge_index, edge_attr, tuple_index, tuple_feat, batch,
           num_graphs, x_emb, ea_emb, tf_emb, W0, b0, W1, b1,
           convW1, convb1, convW2, convb2, predW, predb):
    G = 128
    ti = tuple_index[0]
    tj = tuple_index[1]
    ti32 = ti.astype(jnp.int32)
    tj32 = tj.astype(jnp.int32)
    xi32 = jnp.take(x, ti, axis=0).astype(jnp.int32)
    xj32 = jnp.take(x, tj, axis=0).astype(jnp.int32)
    tf32 = tuple_feat.astype(jnp.int32)
    ea32 = edge_attr.astype(jnp.int32)
    T0 = x_emb @ W0 + b0
    T1 = x_emb @ W1 + b1

    Xval = _sc_tupleinit(xi32, xj32, tf32, T0, T1, tf_emb)
    eaF = _sc_gather_rows(ea32, ea_emb)

    layer_call = pl.pallas_call(
        _layer_body,
        grid=(NEB,),
        in_specs=[_erow_spec(), _erow_spec(), _erow_spec(), _erow_spec(),
                  _full((H, H)), _full((1, H)),
                  _full((H, H)), _full((1, H))],
        out_specs=_erow_spec(),
        out_shape=jax.ShapeDtypeStruct((E, H), jnp.float32),
    )

    for l in range(convW1.shape[0]):
        P = _sc_segsum(tj32, Xval)
        g0 = _sc_gather_rows(ti32, P[0])
        g1 = _sc_gather_rows(ti32, P[1])
        Xval = layer_call(g0, g1, eaF, Xval, convW1[l],
                          convb1[l].reshape(1, H), convW2[l],
                          convb2[l].reshape(1, H))

    xn = jax.ops.segment_max(Xval, ti, num_segments=N)
    xn = jnp.where(jnp.isfinite(xn), xn, 0.0)

    seg_valid = (jnp.arange(G)[:, None] < num_graphs).astype(xn.dtype)
    hsum = jax.ops.segment_sum(xn, batch, num_segments=G) * seg_valid
    cnt = jax.ops.segment_sum(jnp.ones((N, 1), dtype=xn.dtype), batch,
                              num_segments=G) * seg_valid
    h_graph = hsum / jnp.maximum(cnt, 1.0)

    out = pl.pallas_call(
        _pred_body,
        in_specs=[pl.BlockSpec((G, H), lambda: (0, 0)),
                  pl.BlockSpec((H, 1), lambda: (0, 0)),
                  pl.BlockSpec((1, 1), lambda: (0, 0))],
        out_specs=pl.BlockSpec((G, 1), lambda: (0, 0)),
        out_shape=jax.ShapeDtypeStruct((G, 1), jnp.float32),
    )(h_graph, predW, predb.reshape(1, 1))
    return out
